# baseline (speedup 1.0000x reference)
import jax
import jax.numpy as jnp
from jax import lax
from jax.experimental import pallas as pl
from jax.experimental.pallas import tpu as pltpu
from jax.experimental.pallas import tpu_sc as plsc

E = 16384
D = 64
L = 16
NC = 2
NS = 16
NW = NC * NS
EPW = E // NW
HALF = EPW // 2
HGROUPS = HALF // L


def _mf_body(usr2, itm2, usr_nid, itm_nid, edges, out,
             eidx_u, eidx_i, nid_u, nid_i, pair_u, pair_i,
             rows_u, rows_i, score, sem):
    wid = lax.axis_index("s") * NC + lax.axis_index("c")
    base = wid * EPW
    lanes = lax.iota(jnp.int32, L)

    for h in range(2):
        hbase = base + h * HALF

        pltpu.sync_copy(edges.at[0, pl.ds(hbase, HALF)], eidx_u)
        pltpu.sync_copy(edges.at[1, pl.ds(hbase, HALF)], eidx_i)

        cu = pltpu.async_copy(usr_nid.at[eidx_u], nid_u, sem)
        ci = pltpu.async_copy(itm_nid.at[eidx_i], nid_i, sem)
        cu.wait()
        ci.wait()

        def split_body(i, _):
            s = pl.ds(i * L, L)
            nu = nid_u[s]
            ni = nid_i[s]
            pair_u[s] = nu >> 1
            pair_i[s] = ni >> 1
            nid_u[s] = (nu & 1) * D
            nid_i[s] = (ni & 1) * D
            return 0

        lax.fori_loop(0, HGROUPS, split_body, 0, unroll=4)

        gu = pltpu.async_copy(usr2.at[pair_u], rows_u, sem)
        gi = pltpu.async_copy(itm2.at[pair_i], rows_i, sem)
        gu.wait()
        gi.wait()

        def group_body(g, _):
            s = pl.ds(g * L, L)
            rowv = g * L + lanes
            cb_u = nid_u[s]
            cb_i = nid_i[s]

            def dim_body(d, acc):
                u = plsc.load_gather(rows_u, [rowv, cb_u + d])
                v = plsc.load_gather(rows_i, [rowv, cb_i + d])
                return acc + u * v

            acc = lax.fori_loop(0, D, dim_body, jnp.zeros((L,), jnp.float32),
                                unroll=8)
            score[pl.ds(h * HALF + g * L, L)] = acc
            return 0

        lax.fori_loop(0, HGROUPS, group_body, 0)

    pltpu.sync_copy(score, out.at[pl.ds(base, EPW)])


@jax.jit
def _mf_sc(usr2, itm2, usr_n_id, itm_n_id, edge_label_index):
    mesh = plsc.VectorSubcoreMesh(core_axis_name="c", subcore_axis_name="s")
    return pl.kernel(
        _mf_body,
        mesh=mesh,
        compiler_params=pltpu.CompilerParams(needs_layout_passes=False),
        out_type=jax.ShapeDtypeStruct((E,), jnp.float32),
        scratch_types=[
            pltpu.VMEM((HALF,), jnp.int32),
            pltpu.VMEM((HALF,), jnp.int32),
            pltpu.VMEM((HALF,), jnp.int32),
            pltpu.VMEM((HALF,), jnp.int32),
            pltpu.VMEM((HALF,), jnp.int32),
            pltpu.VMEM((HALF,), jnp.int32),
            pltpu.VMEM((HALF, 2 * D), jnp.float32),
            pltpu.VMEM((HALF, 2 * D), jnp.float32),
            pltpu.VMEM((EPW,), jnp.float32),
            pltpu.SemaphoreType.DMA,
        ],
    )(usr2, itm2, usr_n_id, itm_n_id, edge_label_index)


def kernel(usr_table, itm_table, usr_n_id, itm_n_id, edge_label_index):
    u2 = usr_table.reshape(-1, 2 * D)
    i2 = itm_table.reshape(-1, 2 * D)
    return _mf_sc(u2, i2, usr_n_id, itm_n_id, edge_label_index)


# speedup vs baseline: 1.4692x; 1.4692x over previous
import jax
import jax.numpy as jnp
from jax import lax
from jax.experimental import pallas as pl
from jax.experimental.pallas import tpu as pltpu
from jax.experimental.pallas import tpu_sc as plsc

E = 16384
D = 64
L = 16
NC = 2
NS = 16
NW = NC * NS
EPW = E // NW
P = 32
NPASS = EPW // P
PGROUPS = P // L
PR = P * 8


def _mf_body(usr1, itm1, usr_nid, itm_nid, edges, out,
             eidx_u, eidx_i, nid_u, nid_i, blk_u, blk_i,
             rows_u, rows_i, score, sem):
    wid = lax.axis_index("s") * NC + lax.axis_index("c")
    base = wid * EPW
    lanes = lax.iota(jnp.int32, L)

    pltpu.sync_copy(edges.at[0, pl.ds(base, EPW)], eidx_u)
    pltpu.sync_copy(edges.at[1, pl.ds(base, EPW)], eidx_i)

    cu = pltpu.async_copy(usr_nid.at[eidx_u], nid_u, sem)
    ci = pltpu.async_copy(itm_nid.at[eidx_i], nid_i, sem)
    cu.wait()
    ci.wait()

    def split_body(i, _):
        s = pl.ds(i * L, L)
        nu = nid_u[s]
        ni = nid_i[s]
        blk_u[s] = nu & ~7
        blk_i[s] = ni & ~7
        nid_u[s] = nu & 7
        nid_i[s] = ni & 7
        return 0

    lax.fori_loop(0, EPW // L, split_body, 0, unroll=4)

    def pass_body(p, _):
        pbase = p * P

        def fire_body(g, _):
            bu = blk_u[pl.ds(pbase + g * L, L)]
            bi = blk_i[pl.ds(pbase + g * L, L)]
            for lane in range(L):
                slot = g * L + lane
                pltpu.async_copy(
                    usr1.at[pl.ds(pl.multiple_of(bu[lane], 8), 8), :],
                    rows_u.at[pl.ds(slot * 8, 8), :], sem)
                pltpu.async_copy(
                    itm1.at[pl.ds(pl.multiple_of(bi[lane], 8), 8), :],
                    rows_i.at[pl.ds(slot * 8, 8), :], sem)
            return 0

        lax.fori_loop(0, PGROUPS, fire_body, 0)

        pltpu.make_async_copy(usr1.at[pl.ds(0, PR), :], rows_u, sem).wait()
        pltpu.make_async_copy(itm1.at[pl.ds(0, PR), :], rows_i, sem).wait()

        def group_body(g, _):
            s = pl.ds(pbase + g * L, L)
            rowv_u = (g * L + lanes) * 8 + nid_u[s]
            rowv_i = (g * L + lanes) * 8 + nid_i[s]

            def dim_body(d, acc):
                dd = jnp.full((L,), d, jnp.int32)
                u = plsc.load_gather(rows_u, [rowv_u, dd])
                v = plsc.load_gather(rows_i, [rowv_i, dd])
                return acc + u * v

            acc = lax.fori_loop(0, D, dim_body, jnp.zeros((L,), jnp.float32),
                                unroll=8)
            score[pl.ds(pbase + g * L, L)] = acc
            return 0

        lax.fori_loop(0, PGROUPS, group_body, 0)
        return 0

    lax.fori_loop(0, NPASS, pass_body, 0)

    pltpu.sync_copy(score, out.at[pl.ds(base, EPW)])


@jax.jit
def _mf_sc(usr1, itm1, usr_n_id, itm_n_id, edge_label_index):
    mesh = plsc.VectorSubcoreMesh(core_axis_name="c", subcore_axis_name="s")
    return pl.kernel(
        _mf_body,
        mesh=mesh,
        compiler_params=pltpu.CompilerParams(needs_layout_passes=False),
        out_type=jax.ShapeDtypeStruct((E,), jnp.float32),
        scratch_types=[
            pltpu.VMEM((EPW,), jnp.int32),
            pltpu.VMEM((EPW,), jnp.int32),
            pltpu.VMEM((EPW,), jnp.int32),
            pltpu.VMEM((EPW,), jnp.int32),
            pltpu.VMEM((EPW,), jnp.int32),
            pltpu.VMEM((EPW,), jnp.int32),
            pltpu.VMEM((PR, D), jnp.float32),
            pltpu.VMEM((PR, D), jnp.float32),
            pltpu.VMEM((EPW,), jnp.float32),
            pltpu.SemaphoreType.DMA,
        ],
    )(usr1, itm1, usr_n_id, itm_n_id, edge_label_index)


def kernel(usr_table, itm_table, usr_n_id, itm_n_id, edge_label_index):
    return _mf_sc(usr_table, itm_table, usr_n_id, itm_n_id, edge_label_index)
